# Initial kernel scaffold; baseline (speedup 1.0000x reference)
import jax
import jax.numpy as jnp
from jax.experimental import pallas as pl


def kernel(x, edge_index, batch, W1, b1, W2, b2, W3, b3, Wlin, blin):
    raise NotImplementedError("write your pallas kernel here")


# speedup vs baseline: 10.5289x; 10.5289x over previous
import dataclasses
import functools

import jax
import jax.numpy as jnp
from jax import lax
from jax.experimental import pallas as pl
from jax.experimental.pallas import tpu as pltpu
from jax.experimental.pallas import tpu_sc as plsc

NUM_CORES = 2
NUM_SUBCORES = 16
NUM_TILES = NUM_CORES * NUM_SUBCORES
LANES = 16
EB = 128

def _mesh():
    return plsc.VectorSubcoreMesh(core_axis_name="c", subcore_axis_name="s")


def _sc_params():
    cp = pltpu.CompilerParams()
    if "needs_layout_passes" in pltpu.CompilerParams.__dataclass_fields__:
        cp = dataclasses.replace(cp, needs_layout_passes=False)
    return cp


def _deg_partials(dst_t, zdeg, np_pad, ch):
    @functools.partial(
        pl.kernel, mesh=_mesh(),
        out_type=jax.ShapeDtypeStruct((NUM_TILES, np_pad), jnp.float32),
        scratch_types=[
            pltpu.VMEM((ch, EB), jnp.int32),
            pltpu.VMEM((np_pad,), jnp.float32),
        ],
        compiler_params=_sc_params(),
    )
    def k(dst_hbm, zdeg_hbm, out_hbm, dst_v, deg_v):
        cid = lax.axis_index("c")
        sid = lax.axis_index("s")
        wid = cid * NUM_SUBCORES + sid
        pltpu.sync_copy(dst_hbm.at[wid], dst_v)
        pltpu.sync_copy(zdeg_hbm, deg_v)
        ones = jnp.ones((LANES,), jnp.float32)

        @pl.loop(0, ch)
        def _(c):
            for j in range(EB // LANES):
                idx = dst_v[c, pl.ds(j * LANES, LANES)]
                plsc.addupdate_scatter(deg_v, [idx], ones)

        pltpu.sync_copy(deg_v, out_hbm.at[wid])

    return k(dst_t, zdeg)


def _edge_aggregate(hp, src_t, dst_t, zrows, np_pad, ch):
    rows_per_tile = np_pad // NUM_SUBCORES

    @functools.partial(
        pl.kernel, mesh=_mesh(),
        out_type=jax.ShapeDtypeStruct((NUM_CORES, np_pad, 128), jnp.float32),
        scratch_types=[
            pltpu.VMEM((ch, EB), jnp.int32),
            pltpu.VMEM((ch, EB), jnp.int32),
            pltpu.VMEM((EB, 128), jnp.float32),
            pltpu.VMEM_SHARED((np_pad, 128), jnp.float32),
        ],
    )
    def k(hp_hbm, src_hbm, dst_hbm, z_hbm, out_hbm, src_v, dst_v, rows_v, acc_sh):
        cid = lax.axis_index("c")
        sid = lax.axis_index("s")
        wid = cid * NUM_SUBCORES + sid
        pltpu.sync_copy(src_hbm.at[wid], src_v)
        pltpu.sync_copy(dst_hbm.at[wid], dst_v)
        pltpu.sync_copy(z_hbm, acc_sh.at[pl.ds(sid * rows_per_tile, rows_per_tile)])
        plsc.subcore_barrier()

        @pl.loop(0, ch)
        def _(c):
            pltpu.sync_copy(hp_hbm.at[src_v.at[c]], rows_v)
            pltpu.sync_copy(rows_v, acc_sh.at[dst_v.at[c]], add=True)

        plsc.subcore_barrier()
        pltpu.sync_copy(
            acc_sh.at[pl.ds(sid * rows_per_tile, rows_per_tile)],
            out_hbm.at[cid, pl.ds(sid * rows_per_tile, rows_per_tile)],
        )

    return k(hp, src_t, dst_t, zrows)


_PREC = jax.lax.Precision.HIGHEST


def _degsum_body(p_ref, o_ref):
    s = jnp.sum(p_ref[...], axis=0, keepdims=True)
    o_ref[...] = jax.lax.rsqrt(s + 1.0)


def _stage1_body(x_ref, w_ref, dinv_ref, o_ref):
    h = jnp.dot(x_ref[...], w_ref[...], preferred_element_type=jnp.float32,
                precision=_PREC)
    o_ref[...] = h * dinv_ref[...]


def _mid_body(p0_ref, p1_ref, hp_ref, dinv_ref, b_ref, w_ref, o_ref):
    agg = p0_ref[...] + p1_ref[...] + hp_ref[...]
    h = jnp.maximum(agg * dinv_ref[...] + b_ref[...], 0.0)
    o_ref[...] = jnp.dot(h, w_ref[...], preferred_element_type=jnp.float32,
                         precision=_PREC) * dinv_ref[...]


def _final_body(p0_ref, p1_ref, hp_ref, dinv_ref, b_ref, batch_ref, wl_ref,
                bl_ref, o_ref):
    agg = p0_ref[...] + p1_ref[...] + hp_ref[...]
    h = jnp.maximum(agg * dinv_ref[...] + b_ref[...], 0.0)
    n = h.shape[0]
    g = o_ref.shape[0]
    gid = jax.lax.broadcasted_iota(jnp.int32, (g, n), 0)
    mask = (gid == batch_ref[...]).astype(jnp.float32)
    cnt = jnp.sum(mask, axis=1, keepdims=True)
    pooled = jnp.dot(mask, h, preferred_element_type=jnp.float32,
                     precision=_PREC) / jnp.maximum(cnt, 1.0)
    logits = jnp.dot(pooled, wl_ref[...], preferred_element_type=jnp.float32,
                     precision=_PREC) + bl_ref[...]
    m = jnp.max(logits, axis=1, keepdims=True)
    lse = jnp.log(jnp.sum(jnp.exp(logits - m), axis=1, keepdims=True)) + m
    o_ref[...] = logits - lse


def _tc(body, out_shape, *args):
    return pl.pallas_call(body, out_shape=out_shape)(*args)


def kernel(x, edge_index, batch, W1, b1, W2, b2, W3, b3, Wlin, blin):
    n, d = x.shape
    h_dim = W1.shape[1]
    g = 64
    c_dim = Wlin.shape[1]
    e = edge_index.shape[1]

    rows_per_tile = -(-(n + 1) // NUM_SUBCORES)
    rows_per_tile = -(-rows_per_tile // 8) * 8
    np_pad = rows_per_tile * NUM_SUBCORES

    ch = -(-e // (NUM_TILES * EB))
    e_pad = NUM_TILES * ch * EB
    src = edge_index[0].astype(jnp.int32)
    dst = edge_index[1].astype(jnp.int32)
    pad = e_pad - e
    src_t = jnp.concatenate([src, jnp.zeros((pad,), jnp.int32)]) \
        .reshape(NUM_TILES, ch, EB)
    dst_t = jnp.concatenate([dst, jnp.full((pad,), n, jnp.int32)]) \
        .reshape(NUM_TILES, ch, EB)

    zdeg = jnp.zeros((np_pad,), jnp.float32)
    zrows = jnp.zeros((rows_per_tile, h_dim), jnp.float32)

    deg_parts = _deg_partials(dst_t, zdeg, np_pad, ch)
    dinv_row = _tc(_degsum_body,
                   jax.ShapeDtypeStruct((1, np_pad), jnp.float32), deg_parts)
    dinv_col = dinv_row.reshape(np_pad, 1)[:n]

    b1r = b1.reshape(1, h_dim)
    b2r = b2.reshape(1, h_dim)
    b3r = b3.reshape(1, h_dim)
    blr = blin.reshape(1, c_dim)
    batch_row = batch.astype(jnp.int32).reshape(1, n)

    hp = _tc(_stage1_body, jax.ShapeDtypeStruct((n, h_dim), jnp.float32),
             x, W1, dinv_col)

    for (b_r, w_next) in ((b1r, W2), (b2r, W3)):
        parts = _edge_aggregate(hp, src_t, dst_t, zrows, np_pad, ch)
        hp = _tc(_mid_body, jax.ShapeDtypeStruct((n, h_dim), jnp.float32),
                 parts[0, :n], parts[1, :n], hp, dinv_col, b_r, w_next)

    parts = _edge_aggregate(hp, src_t, dst_t, zrows, np_pad, ch)
    out = _tc(_final_body, jax.ShapeDtypeStruct((g, c_dim), jnp.float32),
              parts[0, :n], parts[1, :n], hp, dinv_col, b3r, batch_row,
              Wlin, blr)
    return out


# speedup vs baseline: 17.4765x; 1.6599x over previous
import dataclasses
import functools

import jax
import jax.numpy as jnp
from jax import lax
from jax.experimental import pallas as pl
from jax.experimental.pallas import tpu as pltpu
from jax.experimental.pallas import tpu_sc as plsc

NUM_CORES = 2
NUM_SUBCORES = 16
NUM_TILES = NUM_CORES * NUM_SUBCORES
LANES = 16
EB = 64

def _mesh():
    return plsc.VectorSubcoreMesh(core_axis_name="c", subcore_axis_name="s")


def _sc_params():
    cp = pltpu.CompilerParams()
    if "needs_layout_passes" in pltpu.CompilerParams.__dataclass_fields__:
        cp = dataclasses.replace(cp, needs_layout_passes=False)
    return cp


def _deg_partials(dst_t, zdeg, np_pad, ch):
    @functools.partial(
        pl.kernel, mesh=_mesh(),
        out_type=jax.ShapeDtypeStruct((NUM_TILES, np_pad), jnp.float32),
        scratch_types=[
            pltpu.VMEM((ch, EB), jnp.int32),
            pltpu.VMEM((np_pad,), jnp.float32),
        ],
        compiler_params=_sc_params(),
    )
    def k(dst_hbm, zdeg_hbm, out_hbm, dst_v, deg_v):
        cid = lax.axis_index("c")
        sid = lax.axis_index("s")
        wid = cid * NUM_SUBCORES + sid
        pltpu.sync_copy(dst_hbm.at[wid], dst_v)
        pltpu.sync_copy(zdeg_hbm, deg_v)
        ones = jnp.ones((LANES,), jnp.float32)

        @pl.loop(0, ch)
        def _(c):
            for j in range(EB // LANES):
                idx = dst_v[c, pl.ds(j * LANES, LANES)]
                plsc.addupdate_scatter(deg_v, [idx], ones)

        pltpu.sync_copy(deg_v, out_hbm.at[wid])

    return k(dst_t, zdeg)


def _edge_aggregate(hp, src_t, dst_t, zrows, np_pad, ch):
    rows_per_tile = np_pad // NUM_SUBCORES

    @functools.partial(
        pl.kernel, mesh=_mesh(),
        out_type=jax.ShapeDtypeStruct((NUM_CORES, np_pad, 128), jnp.float32),
        scratch_types=[
            pltpu.VMEM((ch * EB,), jnp.int32),
            pltpu.VMEM((ch, EB), jnp.int32),
            pltpu.VMEM((EB, 128), jnp.float32),
            pltpu.VMEM((EB, 128), jnp.float32),
            pltpu.VMEM_SHARED((np_pad, 128), jnp.float32),
            pltpu.SemaphoreType.DMA,
            pltpu.SemaphoreType.DMA,
        ],
    )
    def k(hp_hbm, src_hbm, dst_hbm, z_hbm, out_hbm, src_v, dst_v,
          rows_a, rows_b, acc_sh, sem_a, sem_b):
        cid = lax.axis_index("c")
        sid = lax.axis_index("s")
        wid = cid * NUM_SUBCORES + sid
        pltpu.sync_copy(src_hbm.at[wid], src_v)
        pltpu.sync_copy(dst_hbm.at[wid], dst_v)

        def gather(c, buf, sem):
            return pltpu.make_async_copy(
                hp_hbm.at[src_v.at[pl.ds(c * EB, EB)]], buf, sem)

        gather(0, rows_a, sem_a).start()
        gather(1, rows_b, sem_b).start()
        pltpu.sync_copy(z_hbm, acc_sh.at[pl.ds(sid * rows_per_tile, rows_per_tile)])
        plsc.subcore_barrier()

        @pl.loop(0, ch - 1, step=2)
        def _(c):
            gather(c, rows_a, sem_a).wait()
            pltpu.sync_copy(rows_a, acc_sh.at[dst_v.at[c]], add=True)

            @pl.when(c + 2 < ch)
            def _():
                gather(c + 2, rows_a, sem_a).start()

            gather(c + 1, rows_b, sem_b).wait()
            pltpu.sync_copy(rows_b, acc_sh.at[dst_v.at[c + 1]], add=True)

            @pl.when(c + 3 < ch)
            def _():
                gather(c + 3, rows_b, sem_b).start()

        if ch % 2 == 1:
            gather(ch - 1, rows_a, sem_a).wait()
            pltpu.sync_copy(rows_a, acc_sh.at[dst_v.at[ch - 1]], add=True)

        plsc.subcore_barrier()
        pltpu.sync_copy(
            acc_sh.at[pl.ds(sid * rows_per_tile, rows_per_tile)],
            out_hbm.at[cid, pl.ds(sid * rows_per_tile, rows_per_tile)],
        )

    return k(hp, src_t, dst_t, zrows)


_PREC = jax.lax.Precision.HIGHEST


def _degsum_body(p_ref, o_ref):
    s = jnp.sum(p_ref[...], axis=0, keepdims=True)
    o_ref[...] = jax.lax.rsqrt(s + 1.0)


def _stage1_body(x_ref, w_ref, dinv_ref, o_ref):
    h = jnp.dot(x_ref[...], w_ref[...], preferred_element_type=jnp.float32,
                precision=_PREC)
    o_ref[...] = h * dinv_ref[...]


def _mid_body(p0_ref, p1_ref, hp_ref, dinv_ref, b_ref, w_ref, o_ref):
    agg = p0_ref[...] + p1_ref[...] + hp_ref[...]
    h = jnp.maximum(agg * dinv_ref[...] + b_ref[...], 0.0)
    o_ref[...] = jnp.dot(h, w_ref[...], preferred_element_type=jnp.float32,
                         precision=_PREC) * dinv_ref[...]


def _final_body(p0_ref, p1_ref, hp_ref, dinv_ref, b_ref, batch_ref, wl_ref,
                bl_ref, o_ref):
    agg = p0_ref[...] + p1_ref[...] + hp_ref[...]
    h = jnp.maximum(agg * dinv_ref[...] + b_ref[...], 0.0)
    n = h.shape[0]
    g = o_ref.shape[0]
    gid = jax.lax.broadcasted_iota(jnp.int32, (g, n), 0)
    mask = (gid == batch_ref[...]).astype(jnp.float32)
    cnt = jnp.sum(mask, axis=1, keepdims=True)
    pooled = jnp.dot(mask, h, preferred_element_type=jnp.float32,
                     precision=_PREC) / jnp.maximum(cnt, 1.0)
    logits = jnp.dot(pooled, wl_ref[...], preferred_element_type=jnp.float32,
                     precision=_PREC) + bl_ref[...]
    m = jnp.max(logits, axis=1, keepdims=True)
    lse = jnp.log(jnp.sum(jnp.exp(logits - m), axis=1, keepdims=True)) + m
    o_ref[...] = logits - lse


def _tc(body, out_shape, *args):
    return pl.pallas_call(body, out_shape=out_shape)(*args)


def kernel(x, edge_index, batch, W1, b1, W2, b2, W3, b3, Wlin, blin):
    n, d = x.shape
    h_dim = W1.shape[1]
    g = 64
    c_dim = Wlin.shape[1]
    e = edge_index.shape[1]

    rows_per_tile = -(-(n + 1) // NUM_SUBCORES)
    rows_per_tile = -(-rows_per_tile // 8) * 8
    np_pad = rows_per_tile * NUM_SUBCORES

    ch = -(-e // (NUM_TILES * EB))
    e_pad = NUM_TILES * ch * EB
    src = edge_index[0].astype(jnp.int32)
    dst = edge_index[1].astype(jnp.int32)
    pad = e_pad - e
    src_t = jnp.concatenate([src, jnp.zeros((pad,), jnp.int32)]) \
        .reshape(NUM_TILES, ch * EB)
    dst_t = jnp.concatenate([dst, jnp.full((pad,), n, jnp.int32)]) \
        .reshape(NUM_TILES, ch, EB)

    zdeg = jnp.zeros((np_pad,), jnp.float32)
    zrows = jnp.zeros((rows_per_tile, h_dim), jnp.float32)

    deg_parts = _deg_partials(dst_t, zdeg, np_pad, ch)
    dinv_row = _tc(_degsum_body,
                   jax.ShapeDtypeStruct((1, np_pad), jnp.float32), deg_parts)
    dinv_col = dinv_row.reshape(np_pad, 1)[:n]

    b1r = b1.reshape(1, h_dim)
    b2r = b2.reshape(1, h_dim)
    b3r = b3.reshape(1, h_dim)
    blr = blin.reshape(1, c_dim)
    batch_row = batch.astype(jnp.int32).reshape(1, n)

    hp = _tc(_stage1_body, jax.ShapeDtypeStruct((n, h_dim), jnp.float32),
             x, W1, dinv_col)

    for (b_r, w_next) in ((b1r, W2), (b2r, W3)):
        parts = _edge_aggregate(hp, src_t, dst_t, zrows, np_pad, ch)
        hp = _tc(_mid_body, jax.ShapeDtypeStruct((n, h_dim), jnp.float32),
                 parts[0, :n], parts[1, :n], hp, dinv_col, b_r, w_next)

    parts = _edge_aggregate(hp, src_t, dst_t, zrows, np_pad, ch)
    out = _tc(_final_body, jax.ShapeDtypeStruct((g, c_dim), jnp.float32),
              parts[0, :n], parts[1, :n], hp, dinv_col, b3r, batch_row,
              Wlin, blr)
    return out
